# baseline (speedup 1.0000x reference)
import functools

import jax
import jax.numpy as jnp
from jax import lax
from jax.experimental import pallas as pl
from jax.experimental.pallas import tpu as pltpu
from jax.experimental.pallas import tpu_sc as plsc

NC = 2
NS = 16
NW = NC * NS
CH = 128
NBUF = 8
H = NBUF // 2


@functools.partial(jax.jit, static_argnames=("n_chunks",))
def _gather_flat(emb, idx3, n_chunks):
    V, D = emb.shape
    per_w = n_chunks * CH
    N = NW * per_w
    groups = n_chunks // NBUF
    assert n_chunks % NBUF == 0 and groups >= 3
    mesh = plsc.VectorSubcoreMesh(
        core_axis_name="c", subcore_axis_name="s", num_cores=NC, num_subcores=NS
    )

    @functools.partial(
        pl.kernel,
        out_type=jax.ShapeDtypeStruct((N, D), jnp.float32),
        mesh=mesh,
        compiler_params=pltpu.CompilerParams(use_tc_tiling_on_sc=False),
        scratch_types=[
            pltpu.VMEM((n_chunks, CH), jnp.int32),
            pltpu.VMEM((NBUF, CH, D), jnp.float32),
        ]
        + [pltpu.SemaphoreType.DMA] * (2 * NBUF),
    )
    def gather_k(emb_hbm, idx_hbm, out_hbm, idx_v, rows_v, *sems):
        gsems = sems[:NBUF]
        ssems = sems[NBUF:]
        wid = lax.axis_index("s") * NC + lax.axis_index("c")
        base = wid * per_w
        pltpu.sync_copy(idx_hbm.at[wid], idx_v)

        def start_gather(j, b):
            pltpu.async_copy(emb_hbm.at[idx_v.at[j]], rows_v.at[b], gsems[b])

        def wait_gather(j, b):
            pltpu.make_async_copy(
                emb_hbm.at[idx_v.at[j]], rows_v.at[b], gsems[b]
            ).wait()

        def start_store(j, b):
            pltpu.async_copy(
                rows_v.at[b], out_hbm.at[pl.ds(base + j * CH, CH)], ssems[b]
            )

        def wait_store(j, b):
            pltpu.make_async_copy(
                rows_v.at[b], out_hbm.at[pl.ds(base + j * CH, CH)], ssems[b]
            ).wait()

        for b in range(NBUF):
            start_gather(b, b)

        for b in range(NBUF):
            wait_gather(b, b)
            start_store(b, b)
            if b >= H:
                bb = b - H
                wait_store(bb, bb)
                start_gather(b + H, bb)

        def body(g, carry):
            for b in range(NBUF):
                j = g * NBUF + b
                wait_gather(j, b)
                start_store(j, b)
                bb = (b - H) % NBUF
                wait_store(j - H, bb)
                start_gather(j + H, bb)
            return carry

        lax.fori_loop(1, groups - 1, body, 0, unroll=False)

        for b in range(NBUF):
            j = (groups - 1) * NBUF + b
            wait_gather(j, b)
            start_store(j, b)
            if b < H:
                bb = (b - H) % NBUF
                wait_store(j - H, bb)
                start_gather(j + H, bb)

        for b in range(NBUF):
            j = (groups - 1) * NBUF + b
            wait_store(j, b)

    return gather_k(emb, idx3)


def kernel(idx, emb):
    B, F = idx.shape
    V, D = emb.shape
    N = B * F
    per_w = N // NW
    n_chunks = per_w // CH
    assert per_w % CH == 0 and N % NW == 0
    idx3 = idx.reshape(NW, n_chunks, CH).astype(jnp.int32)
    out = _gather_flat(emb, idx3, n_chunks)
    return out.reshape(B, F, D)
